# Initial kernel scaffold; baseline (speedup 1.0000x reference)
import jax
import jax.numpy as jnp
from jax.experimental import pallas as pl


def kernel(x, pos, params, edge_index, batch):
    raise NotImplementedError("write your pallas kernel here")


# speedup vs baseline: 168.2006x; 168.2006x over previous
import functools

import jax
import jax.numpy as jnp
from jax import lax
from jax.experimental import pallas as pl
from jax.experimental.pallas import tpu as pltpu
from jax.experimental.pallas import tpu_sc as plsc

R_MAX = 10.0
SH = 9
EMB = 32


def _pick_block(n, candidates):
    for c in candidates:
        if n % c == 0:
            return c
    return n


def _sc_gather(table, idx):
    n = idx.shape[0]
    width = table.shape[1]
    nw = 32
    per_w = n // nw
    chunk = _pick_block(per_w, [400 if width <= 128 else 200, 200, 100, 50, 8])
    mesh = plsc.VectorSubcoreMesh(core_axis_name="c", subcore_axis_name="s")

    @functools.partial(
        pl.kernel,
        out_type=jax.ShapeDtypeStruct((n, width), jnp.float32),
        mesh=mesh,
        scratch_types=[
            pltpu.VMEM((chunk,), jnp.int32),
            pltpu.VMEM((chunk, width), jnp.float32),
            pltpu.SemaphoreType.DMA,
        ],
    )
    def g(tab_hbm, idx_hbm, out_hbm, idx_v, rows_v, sem):
        wid = lax.axis_index("s") * 2 + lax.axis_index("c")
        base = wid * per_w

        def body(k, carry):
            off = base + k * chunk
            pltpu.sync_copy(idx_hbm.at[pl.ds(off, chunk)], idx_v)
            pltpu.async_copy(tab_hbm.at[idx_v], rows_v, sem).wait()
            pltpu.sync_copy(rows_v, out_hbm.at[pl.ds(off, chunk)])
            return carry

        lax.fori_loop(0, per_w // chunk, body, 0)

    return g(table, idx)


def _sc_scatter_add(msg, dst, n_nodes):
    n_edges, width = msg.shape
    nh = width // 128
    per_sc = n_edges // 2
    per_w = per_sc // 16
    rows_per_sub = (n_nodes // 16) // 8 * 8
    tail = n_nodes - 16 * rows_per_sub
    zeros = jnp.zeros((max(rows_per_sub, tail), 128), jnp.float32)
    chunk = _pick_block(per_w, [200, 100, 50, 8])
    mesh = plsc.VectorSubcoreMesh(core_axis_name="c", subcore_axis_name="s")

    @functools.partial(
        pl.kernel,
        out_type=jax.ShapeDtypeStruct((2, n_nodes, width), jnp.float32),
        mesh=mesh,
        scratch_types=[
            pltpu.VMEM((chunk,), jnp.int32),
            pltpu.VMEM((chunk, 128), jnp.float32),
            pltpu.VMEM_SHARED((n_nodes, 128), jnp.float32),
            pltpu.SemaphoreType.DMA,
        ],
    )
    def s(msg_hbm, dst_hbm, zeros_hbm, out_hbm, idx_v, buf_v, acc_sh, sem):
        cid = lax.axis_index("c")
        sid = lax.axis_index("s")
        base = cid * per_sc + sid * per_w
        for h in range(nh):
            pltpu.sync_copy(
                zeros_hbm.at[pl.ds(0, rows_per_sub)],
                acc_sh.at[pl.ds(sid * rows_per_sub, rows_per_sub)])
            if tail:
                @pl.when(sid == 0)
                def _():
                    pltpu.sync_copy(
                        zeros_hbm.at[pl.ds(0, tail)],
                        acc_sh.at[pl.ds(16 * rows_per_sub, tail)])
            plsc.subcore_barrier()

            def body(k, carry):
                off = base + k * chunk
                pltpu.sync_copy(dst_hbm.at[pl.ds(off, chunk)], idx_v)
                pltpu.sync_copy(
                    msg_hbm.at[pl.ds(off, chunk), pl.ds(h * 128, 128)], buf_v)
                pltpu.sync_copy(buf_v, acc_sh.at[idx_v], add=True)
                return carry

            lax.fori_loop(0, per_w // chunk, body, 0)
            plsc.subcore_barrier()
            pltpu.sync_copy(
                acc_sh.at[pl.ds(sid * rows_per_sub, rows_per_sub)],
                out_hbm.at[cid, pl.ds(sid * rows_per_sub, rows_per_sub),
                           pl.ds(h * 128, 128)],
            )
            if tail:
                @pl.when(sid == 0)
                def _():
                    pltpu.sync_copy(
                        acc_sh.at[pl.ds(16 * rows_per_sub, tail)],
                        out_hbm.at[cid, pl.ds(16 * rows_per_sub, tail),
                                   pl.ds(h * 128, 128)],
                    )
            plsc.subcore_barrier()

    return s(msg, dst, zeros)


def _embed(x, w, b, pos8):
    n, dft = x.shape
    emb = w.shape[1]
    bn = _pick_block(n, [2000, 1000, 500, 8])

    def body(x_ref, w_ref, b_ref, p_ref, o_ref):
        h = (jnp.dot(x_ref[...], w_ref[...], preferred_element_type=jnp.float32)
             + b_ref[0:1, :])
        o_ref[...] = jnp.concatenate(
            [h, p_ref[...], jnp.zeros((bn, 128 - emb - 8), jnp.float32)], axis=1)

    return pl.pallas_call(
        body,
        grid=(n // bn,),
        in_specs=[
            pl.BlockSpec((bn, dft), lambda i: (i, 0)),
            pl.BlockSpec((dft, emb), lambda i: (0, 0)),
            pl.BlockSpec((8, emb), lambda i: (0, 0)),
            pl.BlockSpec((bn, 8), lambda i: (i, 0)),
        ],
        out_specs=pl.BlockSpec((bn, 128), lambda i: (i, 0)),
        out_shape=jax.ShapeDtypeStruct((n, 128), jnp.float32),
    )(x, w, b, pos8)


def _sph_rbf(a, b):
    be = a.shape[0]
    v = a[:, EMB:EMB + 16] - b[:, EMB:EMB + 16]
    d = jnp.sqrt(jnp.sum(v * v, axis=1, keepdims=True) + 1e-9)
    inv = 1.0 / d
    x = jnp.broadcast_to(v[:, 0:1] * inv, (be, 16))
    y = jnp.broadcast_to(v[:, 1:2] * inv, (be, 16))
    z = jnp.broadcast_to(v[:, 2:3] * inv, (be, 16))
    li = lax.broadcasted_iota(jnp.int32, (1, 16), 1)
    one = jnp.ones((be, 16), jnp.float32)
    c1 = 3.0 ** 0.5
    c2 = 15.0 ** 0.5
    c3 = 5.0 ** 0.5 / 2.0
    c4 = 15.0 ** 0.5 / 2.0

    def lane_const(vals):
        c = jnp.zeros((1, 16), jnp.float32)
        for j, val in enumerate(vals):
            c = jnp.where(li == j, val, c)
        return c

    sa = jnp.where((li == 2) | (li == 5), y,
                   jnp.where((li == 3) | (li == 6), z,
                             jnp.where(li == 0, one, x)))
    sb = jnp.where(li == 4, y,
                   jnp.where((li == 5) | (li == 6) | (li == 7), z,
                             jnp.where(li == 8, x, one)))
    sc = jnp.where(li == 8, y, one)
    ca = lane_const([1.0, c1, c1, c1, c2, c2, 3.0 * c3, c2, c4])
    cb = lane_const([0.0, 0.0, 0.0, 0.0, 0.0, 0.0, -c3, 0.0, -c4])
    ea = ca * sa * sb + cb * sc * sc
    nb = 16
    wdt = R_MAX / (nb - 1)
    centers = li.astype(jnp.float32) * wdt
    t = (d - centers) * (1.0 / wdt)
    rbf = jnp.exp(-(t * t))
    return ea, rbf


def _tp_msg(hsf, eaf, wtp_refs, dk, dm, packs):
    m = hsf.shape[0]
    msg = None
    s0 = 0
    for g, pack in enumerate(packs):
        t = jnp.dot(hsf[:, :dk], wtp_refs[g][...],
                    preferred_element_type=jnp.float32)
        if pack == 1:
            r = t * eaf[:, s0:s0 + 1]
        else:
            eg = jnp.concatenate(
                [jnp.broadcast_to(eaf[:, s0 + j:s0 + j + 1], (m, dm))
                 for j in range(pack)], axis=1)
            p = t * eg
            r = p[:, :dm]
            for j in range(1, pack):
                r = r + p[:, j * dm:(j + 1) * dm]
        msg = r if msg is None else msg + r
        s0 += pack
    return msg


def _edge_layer(hs2, w1, b1, w2, b2, wtps, packs, dout, dk, dm, dw,
                ea=None, rbf=None):
    ne = hs2.shape[1]
    din_w = hs2.shape[2]
    first = ea is None
    nwt = len(wtps)
    be = _pick_block(ne, [2000, 1600, 800, 400, 8])

    def body(*refs):
        if first:
            (hs_ref, w1_ref, b1_ref, w2_ref, b2_ref), rest = refs[:5], refs[5:]
            wtp_refs = rest[:nwt]
            o_ref, ea_ref, rbf_ref = rest[nwt:]
        else:
            (hs_ref, ea_ref, rbf_ref, w1_ref, b1_ref, w2_ref, b2_ref), rest = \
                refs[:7], refs[7:]
            wtp_refs = rest[:nwt]
            o_ref = rest[nwt]
        a = hs_ref[0]
        b = hs_ref[1]
        if first:
            ea_b, rbf_b = _sph_rbf(a, b)
            ea_ref[...] = ea_b
            rbf_ref[...] = rbf_b
        else:
            ea_b = ea_ref[...]
            rbf_b = rbf_ref[...]
        a32 = a[:, :EMB]
        b32 = b[:, :EMB]
        w1f = w1_ref[...]
        r_pre = (jnp.dot(rbf_b, w1f[:16], preferred_element_type=jnp.float32)
                 + b1_ref[0:1, :])
        asrc = jnp.dot(a32, w1f[16:48], preferred_element_type=jnp.float32)
        adst = jnp.dot(a32, w1f[48:80], preferred_element_type=jnp.float32)
        bsrc = jnp.dot(b32, w1f[16:48], preferred_element_type=jnp.float32)
        bdst = jnp.dot(b32, w1f[48:80], preferred_element_type=jnp.float32)
        hid = jnp.maximum(
            jnp.concatenate([r_pre + asrc + bdst, r_pre + bsrc + adst],
                            axis=0), 0.0)
        w = (jnp.dot(hid, w2_ref[...], preferred_element_type=jnp.float32)
             + b2_ref[0:1, :])
        li16 = lax.broadcasted_iota(jnp.int32, (1, 16), 1)
        sgn = jnp.where((li16 >= 1) & (li16 <= 3), -1.0, 1.0)
        eaf = jnp.concatenate([ea_b, ea_b * sgn], axis=0)
        hsf = jnp.concatenate([a, b], axis=0)
        msg = _tp_msg(hsf, eaf, wtp_refs, dk, dm, packs) * w
        if first:
            lio = lax.broadcasted_iota(jnp.int32, (1, dm), 1)
            msg = jnp.where(lio == dout, 1.0, jnp.where(lio > dout, 0.0, msg))
        if dm < dw:
            msg = jnp.concatenate(
                [msg, jnp.zeros((2 * be, dw - dm), jnp.float32)], axis=1)
        o_ref[0] = msg[:be]
        o_ref[1] = msg[be:]

    wspecs = [
        pl.BlockSpec(w1.shape, lambda i: (0, 0)),
        pl.BlockSpec(b1.shape, lambda i: (0, 0)),
        pl.BlockSpec(w2.shape, lambda i: (0, 0)),
        pl.BlockSpec(b2.shape, lambda i: (0, 0)),
    ] + [pl.BlockSpec(wt.shape, lambda i: (0, 0)) for wt in wtps]
    hspec = pl.BlockSpec((2, be, din_w), lambda i: (0, i, 0))
    gspecs = [
        pl.BlockSpec((be, 16), lambda i: (i, 0)),
        pl.BlockSpec((be, 16), lambda i: (i, 0)),
    ]
    mspec = pl.BlockSpec((2, be, dw), lambda i: (0, i, 0))
    mshape = jax.ShapeDtypeStruct((2, ne, dw), jnp.float32)
    gshapes = [
        jax.ShapeDtypeStruct((ne, 16), jnp.float32),
        jax.ShapeDtypeStruct((ne, 16), jnp.float32),
    ]
    if first:
        return pl.pallas_call(
            body, grid=(ne // be,),
            in_specs=[hspec] + wspecs,
            out_specs=[mspec] + gspecs,
            out_shape=[mshape] + gshapes,
        )(hs2, w1, b1, w2, b2, *wtps)
    return pl.pallas_call(
        body, grid=(ne // be,),
        in_specs=[hspec] + gspecs + wspecs,
        out_specs=mspec,
        out_shape=mshape,
    )(hs2, ea, rbf, w1, b1, w2, b2, *wtps)


def _norm_stats(acc, dout, cinv):
    _, n, d = acc.shape
    bn = _pick_block(n, [2000, 1000, 500, 8])
    nblk = n // bn
    first = cinv is None

    def body(acc_ref, *refs):
        if first:
            agg_ref, st_ref, cinv_ref, acc_scr = refs
        else:
            cinv_in, agg_ref, st_ref, cinv_ref, acc_scr = refs
        i = pl.program_id(0)
        s = acc_ref[0] + acc_ref[1]
        if first:
            cnt = jnp.maximum(s[:, dout:dout + 1], 1.0)
            inv = 1.0 / cnt
            lio = lax.broadcasted_iota(jnp.int32, (1, d), 1)
            agg = jnp.where(lio < dout, s * inv, 0.0)
        else:
            inv = cinv_in[:, 0:1]
            agg = s * inv
        agg_ref[...] = agg
        cinv_ref[...] = jnp.broadcast_to(inv, (bn, 8))

        @pl.when(i == 0)
        def _():
            acc_scr[...] = jnp.zeros_like(acc_scr)

        acc_scr[0:1, :] += jnp.sum(agg, axis=0, keepdims=True)
        acc_scr[1:2, :] += jnp.sum(agg * agg, axis=0, keepdims=True)

        @pl.when(i == nblk - 1)
        def _():
            st_ref[...] = acc_scr[...]

    in_specs = [pl.BlockSpec((2, bn, d), lambda i: (0, i, 0))]
    args = [acc]
    if not first:
        in_specs.append(pl.BlockSpec((bn, 8), lambda i: (i, 0)))
        args.append(cinv)
    return pl.pallas_call(
        body,
        grid=(nblk,),
        in_specs=in_specs,
        out_specs=[
            pl.BlockSpec((bn, d), lambda i: (i, 0)),
            pl.BlockSpec((8, d), lambda i: (0, 0)),
            pl.BlockSpec((bn, 8), lambda i: (i, 0)),
        ],
        out_shape=[
            jax.ShapeDtypeStruct((n, d), jnp.float32),
            jax.ShapeDtypeStruct((8, d), jnp.float32),
            jax.ShapeDtypeStruct((n, 8), jnp.float32),
        ],
        scratch_shapes=[pltpu.VMEM((8, d), jnp.float32)],
    )(*args)


def _norm_apply(agg, stats, gb, h_old, dres, dnew):
    n, d = agg.shape
    dold = h_old.shape[1]
    bn = _pick_block(n, [2000, 1000, 500, 8])
    rn = 1.0 / n

    def body(agg_ref, st_ref, gb_ref, h_ref, o_ref):
        mean = st_ref[0:1, :] * rn
        var = st_ref[1:2, :] * rn - mean * mean
        inv = lax.rsqrt(var + 1e-5)
        hu = (agg_ref[...] - mean) * inv * gb_ref[0:1, :] + gb_ref[1:2, :]
        res = h_ref[:, :dres]
        if dres < dnew:
            res = jnp.concatenate(
                [res, jnp.zeros((bn, dnew - dres), jnp.float32)], axis=1)
        o_ref[...] = hu[:, :dnew] + res

    return pl.pallas_call(
        body,
        grid=(n // bn,),
        in_specs=[
            pl.BlockSpec((bn, d), lambda i: (i, 0)),
            pl.BlockSpec((8, d), lambda i: (0, 0)),
            pl.BlockSpec((8, d), lambda i: (0, 0)),
            pl.BlockSpec((bn, dold), lambda i: (i, 0)),
        ],
        out_specs=pl.BlockSpec((bn, dnew), lambda i: (i, 0)),
        out_shape=jax.ShapeDtypeStruct((n, dnew), jnp.float32),
    )(agg, stats, gb, h_old)


def _pool(h, batch, num_graphs):
    n, d = h.shape
    bn = _pick_block(n, [2000, 1000, 500, 8])
    nblk = n // bn
    b3 = batch.reshape(nblk, 1, bn)

    def body(h_ref, b_ref, o_ref):
        i = pl.program_id(0)
        b = b_ref[0]
        gi = lax.broadcasted_iota(jnp.int32, (num_graphs, 1), 0)
        oh = (b == gi).astype(jnp.float32)
        contrib = jnp.dot(oh, h_ref[...], preferred_element_type=jnp.float32)

        @pl.when(i == 0)
        def _():
            o_ref[...] = jnp.zeros_like(o_ref)

        o_ref[...] += contrib

    return pl.pallas_call(
        body,
        grid=(nblk,),
        in_specs=[
            pl.BlockSpec((bn, d), lambda i: (i, 0)),
            pl.BlockSpec((1, 1, bn), lambda i: (i, 0, 0)),
        ],
        out_specs=pl.BlockSpec((num_graphs, d), lambda i: (0, 0)),
        out_shape=jax.ShapeDtypeStruct((num_graphs, d), jnp.float32),
    )(h, b3)


def _prep_wtp(wtp_raw, dk, dm, packs):
    din, _, dout = wtp_raw.shape
    w = jnp.transpose(wtp_raw, (1, 0, 2))
    w = jnp.pad(w, ((0, 0), (0, dk - din), (0, dm - dout)))
    out = []
    s0 = 0
    for pack in packs:
        g = w[s0:s0 + pack]
        out.append(jnp.transpose(g, (1, 0, 2)).reshape(dk, pack * dm))
        s0 += pack
    assert s0 == SH
    return out


def kernel(x, pos, params, edge_index, batch):
    n_nodes = x.shape[0]
    ne = edge_index.shape[1]
    num_graphs = 8

    ei0 = edge_index[0].astype(jnp.int32)
    ei1 = edge_index[1].astype(jnp.int32)
    eif = jnp.concatenate([ei0, ei1])
    dstf = jnp.concatenate([ei1, ei0])

    pos8 = jnp.pad(pos, ((0, 0), (0, 8 - pos.shape[1])))
    b_emb = jnp.zeros((8, EMB), jnp.float32).at[0].set(params["b_emb"])
    h = _embed(x, params["W_emb"], b_emb, pos8)

    layer_cfg = [
        (EMB, 160, 256, (1,) * SH, EMB, 256),
        (160, 160, 256, (1,) * SH, 256, 256),
        (160, 32, 128, (1,) * SH, EMB, EMB),
    ]
    cinv = None
    ea = rbf = None
    for li, (lp, (dk, dm, dw, packs, dres, dnew)) in enumerate(
            zip(params["layers"], layer_cfg)):
        dout = lp["Wtp"].shape[2]
        hs2 = _sc_gather(h, eif).reshape(2, ne, h.shape[1])

        w1 = lp["W1"]
        b1 = jnp.zeros((8, w1.shape[1]), jnp.float32).at[0].set(lp["b1"])
        w2 = jnp.pad(lp["W2"], ((0, 0), (0, dm - dout)))
        b2 = jnp.zeros((8, dm), jnp.float32).at[0].set(
            jnp.pad(lp["b2"], (0, dm - dout)))
        wtps = _prep_wtp(lp["Wtp"], dk, dm, packs)

        if li == 0:
            msg, ea, rbf = _edge_layer(hs2, w1, b1, w2, b2, wtps, packs,
                                       dout, dk, dm, dw)
        else:
            msg = _edge_layer(hs2, w1, b1, w2, b2, wtps, packs,
                              dout, dk, dm, dw, ea=ea, rbf=rbf)
        acc = _sc_scatter_add(msg.reshape(2 * ne, dw), dstf, n_nodes)
        agg, stats, cinv = _norm_stats(acc, dout, cinv)

        gb = jnp.zeros((8, dw), jnp.float32)
        gb = gb.at[0, :dout].set(lp["gamma"])
        gb = gb.at[1, :dout].set(lp["beta"])
        h = _norm_apply(agg, stats, gb, h, dres, dnew)

    node_embedding = h
    graph_embedding = _pool(h, batch.astype(jnp.int32), num_graphs)
    return node_embedding, graph_embedding


# speedup vs baseline: 211.2900x; 1.2562x over previous
import functools

import jax
import jax.numpy as jnp
from jax import lax
from jax.experimental import pallas as pl
from jax.experimental.pallas import tpu as pltpu
from jax.experimental.pallas import tpu_sc as plsc

R_MAX = 10.0
SH = 9
EMB = 32


def _pick_block(n, candidates):
    for c in candidates:
        if n % c == 0:
            return c
    return n


def _sc_gather(table, idx):
    n = idx.shape[0]
    width = table.shape[1]
    dt = table.dtype
    nw = 32
    per_w = n // nw
    chunk = _pick_block(per_w, [400, 200, 100, 50, 16])
    mesh = plsc.VectorSubcoreMesh(core_axis_name="c", subcore_axis_name="s")

    @functools.partial(
        pl.kernel,
        out_type=jax.ShapeDtypeStruct((n, width), dt),
        mesh=mesh,
        scratch_types=[
            pltpu.VMEM((chunk,), jnp.int32),
            pltpu.VMEM((chunk, width), dt),
            pltpu.SemaphoreType.DMA,
        ],
    )
    def g(tab_hbm, idx_hbm, out_hbm, idx_v, rows_v, sem):
        wid = lax.axis_index("s") * 2 + lax.axis_index("c")
        base = wid * per_w

        def body(k, carry):
            off = base + k * chunk
            pltpu.sync_copy(idx_hbm.at[pl.ds(off, chunk)], idx_v)
            pltpu.async_copy(tab_hbm.at[idx_v], rows_v, sem).wait()
            pltpu.sync_copy(rows_v, out_hbm.at[pl.ds(off, chunk)])
            return carry

        lax.fori_loop(0, per_w // chunk, body, 0)

    return g(table, idx)


def _sc_scatter_add(msg, dst, n_nodes):
    n_edges, width = msg.shape
    nh = width // 128
    per_sc = n_edges // 2
    per_w = per_sc // 16
    rows_per_sub = (n_nodes // 16) // 8 * 8
    tail = n_nodes - 16 * rows_per_sub
    zeros = jnp.zeros((max(rows_per_sub, tail), 128), jnp.float32)
    chunk = _pick_block(per_w, [200, 100, 50, 8])
    mesh = plsc.VectorSubcoreMesh(core_axis_name="c", subcore_axis_name="s")

    @functools.partial(
        pl.kernel,
        out_type=jax.ShapeDtypeStruct((2, n_nodes, width), jnp.float32),
        mesh=mesh,
        scratch_types=[
            pltpu.VMEM((chunk,), jnp.int32),
            pltpu.VMEM((chunk, 128), jnp.float32),
            pltpu.VMEM_SHARED((n_nodes, 128), jnp.float32),
            pltpu.SemaphoreType.DMA,
        ],
    )
    def s(msg_hbm, dst_hbm, zeros_hbm, out_hbm, idx_v, buf_v, acc_sh, sem):
        cid = lax.axis_index("c")
        sid = lax.axis_index("s")
        base = cid * per_sc + sid * per_w
        for h in range(nh):
            pltpu.sync_copy(
                zeros_hbm.at[pl.ds(0, rows_per_sub)],
                acc_sh.at[pl.ds(sid * rows_per_sub, rows_per_sub)])
            if tail:
                @pl.when(sid == 0)
                def _():
                    pltpu.sync_copy(
                        zeros_hbm.at[pl.ds(0, tail)],
                        acc_sh.at[pl.ds(16 * rows_per_sub, tail)])
            plsc.subcore_barrier()

            def body(k, carry):
                off = base + k * chunk
                pltpu.sync_copy(dst_hbm.at[pl.ds(off, chunk)], idx_v)
                pltpu.sync_copy(
                    msg_hbm.at[pl.ds(off, chunk), pl.ds(h * 128, 128)], buf_v)
                pltpu.sync_copy(buf_v, acc_sh.at[idx_v], add=True)
                return carry

            lax.fori_loop(0, per_w // chunk, body, 0)
            plsc.subcore_barrier()
            pltpu.sync_copy(
                acc_sh.at[pl.ds(sid * rows_per_sub, rows_per_sub)],
                out_hbm.at[cid, pl.ds(sid * rows_per_sub, rows_per_sub),
                           pl.ds(h * 128, 128)],
            )
            if tail:
                @pl.when(sid == 0)
                def _():
                    pltpu.sync_copy(
                        acc_sh.at[pl.ds(16 * rows_per_sub, tail)],
                        out_hbm.at[cid, pl.ds(16 * rows_per_sub, tail),
                                   pl.ds(h * 128, 128)],
                    )
            plsc.subcore_barrier()

    return s(msg, dst, zeros)


def _embed(x, w, b, pos8):
    n, dft = x.shape
    emb = w.shape[1]
    bn = _pick_block(n, [2000, 1000, 500, 8])

    def body(x_ref, w_ref, b_ref, p_ref, o_ref):
        h = (jnp.dot(x_ref[...], w_ref[...], preferred_element_type=jnp.float32)
             + b_ref[0:1, :])
        o_ref[...] = jnp.concatenate(
            [h, p_ref[...], jnp.zeros((bn, 128 - emb - 8), jnp.float32)], axis=1)

    return pl.pallas_call(
        body,
        grid=(n // bn,),
        in_specs=[
            pl.BlockSpec((bn, dft), lambda i: (i, 0)),
            pl.BlockSpec((dft, emb), lambda i: (0, 0)),
            pl.BlockSpec((8, emb), lambda i: (0, 0)),
            pl.BlockSpec((bn, 8), lambda i: (i, 0)),
        ],
        out_specs=pl.BlockSpec((bn, 128), lambda i: (i, 0)),
        out_shape=jax.ShapeDtypeStruct((n, 128), jnp.float32),
    )(x, w, b, pos8)


def _sph_rbf(a, b):
    be = a.shape[0]
    v = a[:, EMB:EMB + 16] - b[:, EMB:EMB + 16]
    d = jnp.sqrt(jnp.sum(v * v, axis=1, keepdims=True) + 1e-9)
    inv = 1.0 / d
    x = jnp.broadcast_to(v[:, 0:1] * inv, (be, 16))
    y = jnp.broadcast_to(v[:, 1:2] * inv, (be, 16))
    z = jnp.broadcast_to(v[:, 2:3] * inv, (be, 16))
    li = lax.broadcasted_iota(jnp.int32, (1, 16), 1)
    one = jnp.ones((be, 16), jnp.float32)
    c1 = 3.0 ** 0.5
    c2 = 15.0 ** 0.5
    c3 = 5.0 ** 0.5 / 2.0
    c4 = 15.0 ** 0.5 / 2.0

    def lane_const(vals):
        c = jnp.zeros((1, 16), jnp.float32)
        for j, val in enumerate(vals):
            c = jnp.where(li == j, val, c)
        return c

    sa = jnp.where((li == 2) | (li == 5), y,
                   jnp.where((li == 3) | (li == 6), z,
                             jnp.where(li == 0, one, x)))
    sb = jnp.where(li == 4, y,
                   jnp.where((li == 5) | (li == 6) | (li == 7), z,
                             jnp.where(li == 8, x, one)))
    sc = jnp.where(li == 8, y, one)
    ca = lane_const([1.0, c1, c1, c1, c2, c2, 3.0 * c3, c2, c4])
    cb = lane_const([0.0, 0.0, 0.0, 0.0, 0.0, 0.0, -c3, 0.0, -c4])
    ea = ca * sa * sb + cb * sc * sc
    nb = 16
    wdt = R_MAX / (nb - 1)
    centers = li.astype(jnp.float32) * wdt
    t = (d - centers) * (1.0 / wdt)
    rbf = jnp.exp(-(t * t))
    return ea, rbf


def _tp_msg(hsf, eaf, wtp_refs, dk, dm, packs):
    m = hsf.shape[0]
    msg = None
    s0 = 0
    for g, pack in enumerate(packs):
        t = jnp.dot(hsf[:, :dk], wtp_refs[g][...],
                    preferred_element_type=jnp.float32)
        if pack == 1:
            r = t * eaf[:, s0:s0 + 1]
        else:
            eg = jnp.concatenate(
                [jnp.broadcast_to(eaf[:, s0 + j:s0 + j + 1], (m, dm))
                 for j in range(pack)], axis=1)
            p = t * eg
            r = p[:, :dm]
            for j in range(1, pack):
                r = r + p[:, j * dm:(j + 1) * dm]
        msg = r if msg is None else msg + r
        s0 += pack
    return msg


def _edge_layer(hs2, w1, b1, w2, b2, wtps, packs, dout, dk, dm, dw,
                ea=None, rbf=None):
    ne = hs2.shape[1]
    din_w = hs2.shape[2]
    first = ea is None
    nwt = len(wtps)
    be = _pick_block(ne, [2000, 1600, 800, 400, 8])

    def body(*refs):
        if first:
            (hs_ref, w1_ref, b1_ref, w2_ref, b2_ref), rest = refs[:5], refs[5:]
            wtp_refs = rest[:nwt]
            o_ref, ea_ref, rbf_ref = rest[nwt:]
        else:
            (hs_ref, ea_ref, rbf_ref, w1_ref, b1_ref, w2_ref, b2_ref), rest = \
                refs[:7], refs[7:]
            wtp_refs = rest[:nwt]
            o_ref = rest[nwt]
        a = hs_ref[0][...].astype(jnp.float32)
        b = hs_ref[1][...].astype(jnp.float32)
        if first:
            ea_b, rbf_b = _sph_rbf(a, b)
            ea_ref[...] = ea_b
            rbf_ref[...] = rbf_b
        else:
            ea_b = ea_ref[...]
            rbf_b = rbf_ref[...]
        a32 = a[:, :EMB]
        b32 = b[:, :EMB]
        w1f = w1_ref[...]
        r_pre = (jnp.dot(rbf_b, w1f[:16], preferred_element_type=jnp.float32)
                 + b1_ref[0:1, :])
        asrc = jnp.dot(a32, w1f[16:48], preferred_element_type=jnp.float32)
        adst = jnp.dot(a32, w1f[48:80], preferred_element_type=jnp.float32)
        bsrc = jnp.dot(b32, w1f[16:48], preferred_element_type=jnp.float32)
        bdst = jnp.dot(b32, w1f[48:80], preferred_element_type=jnp.float32)
        hid = jnp.maximum(
            jnp.concatenate([r_pre + asrc + bdst, r_pre + bsrc + adst],
                            axis=0), 0.0)
        w = (jnp.dot(hid, w2_ref[...], preferred_element_type=jnp.float32)
             + b2_ref[0:1, :])
        li16 = lax.broadcasted_iota(jnp.int32, (1, 16), 1)
        sgn = jnp.where((li16 >= 1) & (li16 <= 3), -1.0, 1.0)
        eaf = jnp.concatenate([ea_b, ea_b * sgn], axis=0)
        hsf = jnp.concatenate([a, b], axis=0)
        msg = _tp_msg(hsf, eaf, wtp_refs, dk, dm, packs) * w
        if first:
            lio = lax.broadcasted_iota(jnp.int32, (1, dm), 1)
            msg = jnp.where(lio == dout, 1.0, jnp.where(lio > dout, 0.0, msg))
        if dm < dw:
            msg = jnp.concatenate(
                [msg, jnp.zeros((2 * be, dw - dm), jnp.float32)], axis=1)
        o_ref[0] = msg[:be]
        o_ref[1] = msg[be:]

    wspecs = [
        pl.BlockSpec(w1.shape, lambda i: (0, 0)),
        pl.BlockSpec(b1.shape, lambda i: (0, 0)),
        pl.BlockSpec(w2.shape, lambda i: (0, 0)),
        pl.BlockSpec(b2.shape, lambda i: (0, 0)),
    ] + [pl.BlockSpec(wt.shape, lambda i: (0, 0)) for wt in wtps]
    hspec = pl.BlockSpec((2, be, din_w), lambda i: (0, i, 0))
    gspecs = [
        pl.BlockSpec((be, 16), lambda i: (i, 0)),
        pl.BlockSpec((be, 16), lambda i: (i, 0)),
    ]
    mspec = pl.BlockSpec((2, be, dw), lambda i: (0, i, 0))
    mshape = jax.ShapeDtypeStruct((2, ne, dw), jnp.float32)
    gshapes = [
        jax.ShapeDtypeStruct((ne, 16), jnp.float32),
        jax.ShapeDtypeStruct((ne, 16), jnp.float32),
    ]
    if first:
        return pl.pallas_call(
            body, grid=(ne // be,),
            in_specs=[hspec] + wspecs,
            out_specs=[mspec] + gspecs,
            out_shape=[mshape] + gshapes,
        )(hs2, w1, b1, w2, b2, *wtps)
    return pl.pallas_call(
        body, grid=(ne // be,),
        in_specs=[hspec] + gspecs + wspecs,
        out_specs=mspec,
        out_shape=mshape,
    )(hs2, ea, rbf, w1, b1, w2, b2, *wtps)


def _norm_stats(accs, dout, cinv):
    nacc = len(accs)
    _, n, d = accs[0].shape
    bn = _pick_block(n, [2000, 1000, 500, 8])
    nblk = n // bn
    first = cinv is None

    def body(*refs):
        acc_refs = refs[:nacc]
        refs = refs[nacc:]
        if first:
            agg_ref, st_ref, cinv_ref, acc_scr = refs
        else:
            cinv_in, agg_ref, st_ref, cinv_ref, acc_scr = refs
        i = pl.program_id(0)
        s = acc_refs[0][0] + acc_refs[0][1]
        for ar in acc_refs[1:]:
            s = s + ar[0] + ar[1]
        if first:
            cnt = jnp.maximum(s[:, dout:dout + 1], 1.0)
            inv = 1.0 / cnt
            lio = lax.broadcasted_iota(jnp.int32, (1, d), 1)
            agg = jnp.where(lio < dout, s * inv, 0.0)
        else:
            inv = cinv_in[:, 0:1]
            agg = s * inv
        agg_ref[...] = agg
        cinv_ref[...] = jnp.broadcast_to(inv, (bn, 8))

        @pl.when(i == 0)
        def _():
            acc_scr[...] = jnp.zeros_like(acc_scr)

        acc_scr[0:1, :] += jnp.sum(agg, axis=0, keepdims=True)
        acc_scr[1:2, :] += jnp.sum(agg * agg, axis=0, keepdims=True)

        @pl.when(i == nblk - 1)
        def _():
            st_ref[...] = acc_scr[...]

    in_specs = [pl.BlockSpec((2, bn, d), lambda i: (0, i, 0))
                for _ in range(nacc)]
    args = list(accs)
    if not first:
        in_specs.append(pl.BlockSpec((bn, 8), lambda i: (i, 0)))
        args.append(cinv)
    return pl.pallas_call(
        body,
        grid=(nblk,),
        in_specs=in_specs,
        out_specs=[
            pl.BlockSpec((bn, d), lambda i: (i, 0)),
            pl.BlockSpec((8, d), lambda i: (0, 0)),
            pl.BlockSpec((bn, 8), lambda i: (i, 0)),
        ],
        out_shape=[
            jax.ShapeDtypeStruct((n, d), jnp.float32),
            jax.ShapeDtypeStruct((8, d), jnp.float32),
            jax.ShapeDtypeStruct((n, 8), jnp.float32),
        ],
        scratch_shapes=[pltpu.VMEM((8, d), jnp.float32)],
    )(*args)


def _norm_apply(agg, stats, gb, h_old, dres, dnew, out_dtype=jnp.float32):
    n, d = agg.shape
    dold = h_old.shape[1]
    bn = _pick_block(n, [2000, 1000, 500, 8])
    rn = 1.0 / n

    def body(agg_ref, st_ref, gb_ref, h_ref, o_ref):
        mean = st_ref[0:1, :] * rn
        var = st_ref[1:2, :] * rn - mean * mean
        inv = lax.rsqrt(var + 1e-5)
        hu = (agg_ref[...] - mean) * inv * gb_ref[0:1, :] + gb_ref[1:2, :]
        res = h_ref[:, :dres].astype(jnp.float32)
        if dres < dnew:
            res = jnp.concatenate(
                [res, jnp.zeros((bn, dnew - dres), jnp.float32)], axis=1)
        o_ref[...] = (hu[:, :dnew] + res).astype(out_dtype)

    return pl.pallas_call(
        body,
        grid=(n // bn,),
        in_specs=[
            pl.BlockSpec((bn, d), lambda i: (i, 0)),
            pl.BlockSpec((8, d), lambda i: (0, 0)),
            pl.BlockSpec((8, d), lambda i: (0, 0)),
            pl.BlockSpec((bn, dold), lambda i: (i, 0)),
        ],
        out_specs=pl.BlockSpec((bn, dnew), lambda i: (i, 0)),
        out_shape=jax.ShapeDtypeStruct((n, dnew), out_dtype),
    )(agg, stats, gb, h_old)


def _pool(h, batch, num_graphs):
    n, d = h.shape
    bn = _pick_block(n, [2000, 1000, 500, 8])
    nblk = n // bn
    b3 = batch.reshape(nblk, 1, bn)

    def body(h_ref, b_ref, o_ref):
        i = pl.program_id(0)
        b = b_ref[0]
        gi = lax.broadcasted_iota(jnp.int32, (num_graphs, 1), 0)
        oh = (b == gi).astype(jnp.float32)
        contrib = jnp.dot(oh, h_ref[...], preferred_element_type=jnp.float32)

        @pl.when(i == 0)
        def _():
            o_ref[...] = jnp.zeros_like(o_ref)

        o_ref[...] += contrib

    return pl.pallas_call(
        body,
        grid=(nblk,),
        in_specs=[
            pl.BlockSpec((bn, d), lambda i: (i, 0)),
            pl.BlockSpec((1, 1, bn), lambda i: (i, 0, 0)),
        ],
        out_specs=pl.BlockSpec((num_graphs, d), lambda i: (0, 0)),
        out_shape=jax.ShapeDtypeStruct((num_graphs, d), jnp.float32),
    )(h, b3)


def _prep_wtp(wtp_raw, dk, dm, packs):
    din, _, dout = wtp_raw.shape
    w = jnp.transpose(wtp_raw, (1, 0, 2))
    w = jnp.pad(w, ((0, 0), (0, dk - din), (0, dm - dout)))
    out = []
    s0 = 0
    for pack in packs:
        g = w[s0:s0 + pack]
        out.append(jnp.transpose(g, (1, 0, 2)).reshape(dk, pack * dm))
        s0 += pack
    assert s0 == SH
    return out


def kernel(x, pos, params, edge_index, batch):
    n_nodes = x.shape[0]
    ne = edge_index.shape[1]
    num_graphs = 8

    ei0 = edge_index[0].astype(jnp.int32)
    ei1 = edge_index[1].astype(jnp.int32)
    nch = 2
    nh = ne // nch
    chunks = []
    for c in range(nch):
        sl = slice(c * nh, (c + 1) * nh)
        chunks.append((jnp.concatenate([ei0[sl], ei1[sl]]),
                       jnp.concatenate([ei1[sl], ei0[sl]])))

    pos8 = jnp.pad(pos, ((0, 0), (0, 8 - pos.shape[1])))
    b_emb = jnp.zeros((8, EMB), jnp.float32).at[0].set(params["b_emb"])
    h = _embed(x, params["W_emb"], b_emb, pos8)

    layer_cfg = [
        (EMB, 160, 256, (1,) * SH, EMB, 256),
        (160, 160, 256, (1,) * SH, 256, 256),
        (160, 32, 128, (1,) * SH, EMB, EMB),
    ]
    cinv = None
    ea_ch = [None] * nch
    rbf_ch = [None] * nch
    for li, (lp, (dk, dm, dw, packs, dres, dnew)) in enumerate(
            zip(params["layers"], layer_cfg)):
        dout = lp["Wtp"].shape[2]
        w1 = lp["W1"]
        b1 = jnp.zeros((8, w1.shape[1]), jnp.float32).at[0].set(lp["b1"])
        w2 = jnp.pad(lp["W2"], ((0, 0), (0, dm - dout)))
        b2 = jnp.zeros((8, dm), jnp.float32).at[0].set(
            jnp.pad(lp["b2"], (0, dm - dout)))
        wtps = _prep_wtp(lp["Wtp"], dk, dm, packs)

        accs = []
        for c, (eif_c, dst_c) in enumerate(chunks):
            hs2 = _sc_gather(h, eif_c).reshape(2, nh, h.shape[1])
            if li == 0:
                msg, ea_ch[c], rbf_ch[c] = _edge_layer(
                    hs2, w1, b1, w2, b2, wtps, packs, dout, dk, dm, dw)
            else:
                msg = _edge_layer(hs2, w1, b1, w2, b2, wtps, packs,
                                  dout, dk, dm, dw, ea=ea_ch[c], rbf=rbf_ch[c])
            accs.append(_sc_scatter_add(msg.reshape(2 * nh, dw), dst_c, n_nodes))
        agg, stats, cinv = _norm_stats(accs, dout, cinv)

        gb = jnp.zeros((8, dw), jnp.float32)
        gb = gb.at[0, :dout].set(lp["gamma"])
        gb = gb.at[1, :dout].set(lp["beta"])
        h = _norm_apply(agg, stats, gb, h, dres, dnew)

    node_embedding = h
    graph_embedding = _pool(h, batch.astype(jnp.int32), num_graphs)
    return node_embedding, graph_embedding


# speedup vs baseline: 223.1144x; 1.0560x over previous
import functools

import jax
import jax.numpy as jnp
from jax import lax
from jax.experimental import pallas as pl
from jax.experimental.pallas import tpu as pltpu
from jax.experimental.pallas import tpu_sc as plsc

R_MAX = 10.0
SH = 9
EMB = 32


def _pick_block(n, candidates):
    for c in candidates:
        if n % c == 0:
            return c
    return n


def _sc_gather(table, idx):
    n = idx.shape[0]
    width = table.shape[1]
    dt = table.dtype
    nw = 32
    per_w = n // nw
    chunk = _pick_block(per_w, [400, 200, 100, 50, 16])
    mesh = plsc.VectorSubcoreMesh(core_axis_name="c", subcore_axis_name="s")

    @functools.partial(
        pl.kernel,
        out_type=jax.ShapeDtypeStruct((n, width), dt),
        mesh=mesh,
        scratch_types=[
            pltpu.VMEM((chunk,), jnp.int32),
            pltpu.VMEM((chunk, width), dt),
            pltpu.SemaphoreType.DMA,
        ],
    )
    def g(tab_hbm, idx_hbm, out_hbm, idx_v, rows_v, sem):
        wid = lax.axis_index("s") * 2 + lax.axis_index("c")
        base = wid * per_w

        def body(k, carry):
            off = base + k * chunk
            pltpu.sync_copy(idx_hbm.at[pl.ds(off, chunk)], idx_v)
            pltpu.async_copy(tab_hbm.at[idx_v], rows_v, sem).wait()
            pltpu.sync_copy(rows_v, out_hbm.at[pl.ds(off, chunk)])
            return carry

        lax.fori_loop(0, per_w // chunk, body, 0)

    return g(table, idx)


def _sc_scatter_add(msg, dst, n_nodes):
    n_edges, width = msg.shape
    nh = width // 128
    per_sc = n_edges // 2
    per_w = per_sc // 16
    rows_per_sub = (n_nodes // 16) // 8 * 8
    tail = n_nodes - 16 * rows_per_sub
    zeros = jnp.zeros((max(rows_per_sub, tail), 128), jnp.float32)
    chunk = _pick_block(per_w, [200, 100, 50, 8])
    mesh = plsc.VectorSubcoreMesh(core_axis_name="c", subcore_axis_name="s")

    @functools.partial(
        pl.kernel,
        out_type=jax.ShapeDtypeStruct((2, n_nodes, width), jnp.float32),
        mesh=mesh,
        scratch_types=[
            pltpu.VMEM((chunk,), jnp.int32),
            pltpu.VMEM((chunk, 128), jnp.float32),
            pltpu.VMEM_SHARED((n_nodes, 128), jnp.float32),
            pltpu.SemaphoreType.DMA,
        ],
    )
    def s(msg_hbm, dst_hbm, zeros_hbm, out_hbm, idx_v, buf_v, acc_sh, sem):
        cid = lax.axis_index("c")
        sid = lax.axis_index("s")
        base = cid * per_sc + sid * per_w
        for h in range(nh):
            pltpu.sync_copy(
                zeros_hbm.at[pl.ds(0, rows_per_sub)],
                acc_sh.at[pl.ds(sid * rows_per_sub, rows_per_sub)])
            if tail:
                @pl.when(sid == 0)
                def _():
                    pltpu.sync_copy(
                        zeros_hbm.at[pl.ds(0, tail)],
                        acc_sh.at[pl.ds(16 * rows_per_sub, tail)])
            plsc.subcore_barrier()

            def body(k, carry):
                off = base + k * chunk
                pltpu.sync_copy(dst_hbm.at[pl.ds(off, chunk)], idx_v)
                pltpu.sync_copy(
                    msg_hbm.at[pl.ds(off, chunk), pl.ds(h * 128, 128)], buf_v)
                pltpu.sync_copy(buf_v, acc_sh.at[idx_v], add=True)
                return carry

            lax.fori_loop(0, per_w // chunk, body, 0)
            plsc.subcore_barrier()
            pltpu.sync_copy(
                acc_sh.at[pl.ds(sid * rows_per_sub, rows_per_sub)],
                out_hbm.at[cid, pl.ds(sid * rows_per_sub, rows_per_sub),
                           pl.ds(h * 128, 128)],
            )
            if tail:
                @pl.when(sid == 0)
                def _():
                    pltpu.sync_copy(
                        acc_sh.at[pl.ds(16 * rows_per_sub, tail)],
                        out_hbm.at[cid, pl.ds(16 * rows_per_sub, tail),
                                   pl.ds(h * 128, 128)],
                    )
            plsc.subcore_barrier()

    return s(msg, dst, zeros)


def _embed(x, w, b, pos8):
    n, dft = x.shape
    emb = w.shape[1]
    bn = _pick_block(n, [2000, 1000, 500, 8])

    def body(x_ref, w_ref, b_ref, p_ref, o_ref):
        h = (jnp.dot(x_ref[...], w_ref[...], preferred_element_type=jnp.float32)
             + b_ref[0:1, :])
        o_ref[...] = jnp.concatenate(
            [h, p_ref[...], jnp.zeros((bn, 128 - emb - 8), jnp.float32)], axis=1)

    return pl.pallas_call(
        body,
        grid=(n // bn,),
        in_specs=[
            pl.BlockSpec((bn, dft), lambda i: (i, 0)),
            pl.BlockSpec((dft, emb), lambda i: (0, 0)),
            pl.BlockSpec((8, emb), lambda i: (0, 0)),
            pl.BlockSpec((bn, 8), lambda i: (i, 0)),
        ],
        out_specs=pl.BlockSpec((bn, 128), lambda i: (i, 0)),
        out_shape=jax.ShapeDtypeStruct((n, 128), jnp.float32),
    )(x, w, b, pos8)


def _sph_rbf(a, b):
    be = a.shape[0]
    v = a[:, EMB:EMB + 16] - b[:, EMB:EMB + 16]
    d = jnp.sqrt(jnp.sum(v * v, axis=1, keepdims=True) + 1e-9)
    inv = 1.0 / d
    x = jnp.broadcast_to(v[:, 0:1] * inv, (be, 16))
    y = jnp.broadcast_to(v[:, 1:2] * inv, (be, 16))
    z = jnp.broadcast_to(v[:, 2:3] * inv, (be, 16))
    li = lax.broadcasted_iota(jnp.int32, (1, 16), 1)
    one = jnp.ones((be, 16), jnp.float32)
    c1 = 3.0 ** 0.5
    c2 = 15.0 ** 0.5
    c3 = 5.0 ** 0.5 / 2.0
    c4 = 15.0 ** 0.5 / 2.0

    def lane_const(vals):
        c = jnp.zeros((1, 16), jnp.float32)
        for j, val in enumerate(vals):
            c = jnp.where(li == j, val, c)
        return c

    sa = jnp.where((li == 2) | (li == 5), y,
                   jnp.where((li == 3) | (li == 6), z,
                             jnp.where(li == 0, one, x)))
    sb = jnp.where(li == 4, y,
                   jnp.where((li == 5) | (li == 6) | (li == 7), z,
                             jnp.where(li == 8, x, one)))
    sc = jnp.where(li == 8, y, one)
    ca = lane_const([1.0, c1, c1, c1, c2, c2, 3.0 * c3, c2, c4])
    cb = lane_const([0.0, 0.0, 0.0, 0.0, 0.0, 0.0, -c3, 0.0, -c4])
    ea = ca * sa * sb + cb * sc * sc
    nb = 16
    wdt = R_MAX / (nb - 1)
    centers = li.astype(jnp.float32) * wdt
    t = (d - centers) * (1.0 / wdt)
    rbf = jnp.exp(-(t * t))
    return ea, rbf


def _tp_msg(hsf, eaf, wtp_refs, dk, dm, packs):
    m = hsf.shape[0]
    msg = None
    s0 = 0
    for g, pack in enumerate(packs):
        t = jnp.dot(hsf[:, :dk], wtp_refs[g][...],
                    preferred_element_type=jnp.float32)
        if pack == 1:
            r = t * eaf[:, s0:s0 + 1]
        else:
            eg = jnp.concatenate(
                [jnp.broadcast_to(eaf[:, s0 + j:s0 + j + 1], (m, dm))
                 for j in range(pack)], axis=1)
            p = t * eg
            r = p[:, :dm]
            for j in range(1, pack):
                r = r + p[:, j * dm:(j + 1) * dm]
        msg = r if msg is None else msg + r
        s0 += pack
    return msg


def _edge_layer(hs2, w1, b1, w2, b2, wtps, packs, dout, dk, dm, dw,
                ea=None, rbf=None):
    ne = hs2.shape[1]
    din_w = hs2.shape[2]
    first = ea is None
    nwt = len(wtps)
    be = _pick_block(ne, [2000, 1600, 800, 400, 8])

    def body(*refs):
        if first:
            (hs_ref, w1_ref, b1_ref, w2_ref, b2_ref), rest = refs[:5], refs[5:]
            wtp_refs = rest[:nwt]
            o_ref, ea_ref, rbf_ref = rest[nwt:]
        else:
            (hs_ref, ea_ref, rbf_ref, w1_ref, b1_ref, w2_ref, b2_ref), rest = \
                refs[:7], refs[7:]
            wtp_refs = rest[:nwt]
            o_ref = rest[nwt]
        a = hs_ref[0][...].astype(jnp.float32)
        b = hs_ref[1][...].astype(jnp.float32)
        if first:
            ea_b, rbf_b = _sph_rbf(a, b)
            ea_ref[...] = ea_b
            rbf_ref[...] = rbf_b
        else:
            ea_b = ea_ref[...]
            rbf_b = rbf_ref[...]
        a32 = a[:, :EMB]
        b32 = b[:, :EMB]
        w1f = w1_ref[...]
        r_pre = (jnp.dot(rbf_b, w1f[:16], preferred_element_type=jnp.float32)
                 + b1_ref[0:1, :])
        asrc = jnp.dot(a32, w1f[16:48], preferred_element_type=jnp.float32)
        adst = jnp.dot(a32, w1f[48:80], preferred_element_type=jnp.float32)
        bsrc = jnp.dot(b32, w1f[16:48], preferred_element_type=jnp.float32)
        bdst = jnp.dot(b32, w1f[48:80], preferred_element_type=jnp.float32)
        hid = jnp.maximum(
            jnp.concatenate([r_pre + asrc + bdst, r_pre + bsrc + adst],
                            axis=0), 0.0)
        w = (jnp.dot(hid, w2_ref[...], preferred_element_type=jnp.float32)
             + b2_ref[0:1, :])
        li16 = lax.broadcasted_iota(jnp.int32, (1, 16), 1)
        sgn = jnp.where((li16 >= 1) & (li16 <= 3), -1.0, 1.0)
        eaf = jnp.concatenate([ea_b, ea_b * sgn], axis=0)
        hsf = jnp.concatenate([a, b], axis=0)
        msg = _tp_msg(hsf, eaf, wtp_refs, dk, dm, packs) * w
        if first:
            lio = lax.broadcasted_iota(jnp.int32, (1, dm), 1)
            msg = jnp.where(lio == dout, 1.0, jnp.where(lio > dout, 0.0, msg))
        if dm < dw:
            msg = jnp.concatenate(
                [msg, jnp.zeros((2 * be, dw - dm), jnp.float32)], axis=1)
        o_ref[0] = msg[:be]
        o_ref[1] = msg[be:]

    wspecs = [
        pl.BlockSpec(w1.shape, lambda i: (0, 0)),
        pl.BlockSpec(b1.shape, lambda i: (0, 0)),
        pl.BlockSpec(w2.shape, lambda i: (0, 0)),
        pl.BlockSpec(b2.shape, lambda i: (0, 0)),
    ] + [pl.BlockSpec(wt.shape, lambda i: (0, 0)) for wt in wtps]
    hspec = pl.BlockSpec((2, be, din_w), lambda i: (0, i, 0))
    gspecs = [
        pl.BlockSpec((be, 16), lambda i: (i, 0)),
        pl.BlockSpec((be, 16), lambda i: (i, 0)),
    ]
    mspec = pl.BlockSpec((2, be, dw), lambda i: (0, i, 0))
    mshape = jax.ShapeDtypeStruct((2, ne, dw), jnp.float32)
    gshapes = [
        jax.ShapeDtypeStruct((ne, 16), jnp.float32),
        jax.ShapeDtypeStruct((ne, 16), jnp.float32),
    ]
    if first:
        return pl.pallas_call(
            body, grid=(ne // be,),
            in_specs=[hspec] + wspecs,
            out_specs=[mspec] + gspecs,
            out_shape=[mshape] + gshapes,
        )(hs2, w1, b1, w2, b2, *wtps)
    return pl.pallas_call(
        body, grid=(ne // be,),
        in_specs=[hspec] + gspecs + wspecs,
        out_specs=mspec,
        out_shape=mshape,
    )(hs2, ea, rbf, w1, b1, w2, b2, *wtps)


def _norm_stats(accs, dout, cinv):
    nacc = len(accs)
    _, n, d = accs[0].shape
    bn = _pick_block(n, [2000, 1000, 500, 8])
    nblk = n // bn
    first = cinv is None

    def body(*refs):
        acc_refs = refs[:nacc]
        refs = refs[nacc:]
        if first:
            agg_ref, st_ref, cinv_ref, acc_scr = refs
        else:
            cinv_in, agg_ref, st_ref, cinv_ref, acc_scr = refs
        i = pl.program_id(0)
        s = acc_refs[0][0] + acc_refs[0][1]
        for ar in acc_refs[1:]:
            s = s + ar[0] + ar[1]
        if first:
            cnt = jnp.maximum(s[:, dout:dout + 1], 1.0)
            inv = 1.0 / cnt
            lio = lax.broadcasted_iota(jnp.int32, (1, d), 1)
            agg = jnp.where(lio < dout, s * inv, 0.0)
        else:
            inv = cinv_in[:, 0:1]
            agg = s * inv
        agg_ref[...] = agg
        cinv_ref[...] = jnp.broadcast_to(inv, (bn, 8))

        @pl.when(i == 0)
        def _():
            acc_scr[...] = jnp.zeros_like(acc_scr)

        acc_scr[0:1, :] += jnp.sum(agg, axis=0, keepdims=True)
        acc_scr[1:2, :] += jnp.sum(agg * agg, axis=0, keepdims=True)

        @pl.when(i == nblk - 1)
        def _():
            st_ref[...] = acc_scr[...]

    in_specs = [pl.BlockSpec((2, bn, d), lambda i: (0, i, 0))
                for _ in range(nacc)]
    args = list(accs)
    if not first:
        in_specs.append(pl.BlockSpec((bn, 8), lambda i: (i, 0)))
        args.append(cinv)
    return pl.pallas_call(
        body,
        grid=(nblk,),
        in_specs=in_specs,
        out_specs=[
            pl.BlockSpec((bn, d), lambda i: (i, 0)),
            pl.BlockSpec((8, d), lambda i: (0, 0)),
            pl.BlockSpec((bn, 8), lambda i: (i, 0)),
        ],
        out_shape=[
            jax.ShapeDtypeStruct((n, d), jnp.float32),
            jax.ShapeDtypeStruct((8, d), jnp.float32),
            jax.ShapeDtypeStruct((n, 8), jnp.float32),
        ],
        scratch_shapes=[pltpu.VMEM((8, d), jnp.float32)],
    )(*args)


def _norm_apply(agg, stats, gb, h_old, dres, dnew, out_dtype=jnp.float32):
    n, d = agg.shape
    dold = h_old.shape[1]
    bn = _pick_block(n, [2000, 1000, 500, 8])
    rn = 1.0 / n

    def body(agg_ref, st_ref, gb_ref, h_ref, o_ref):
        mean = st_ref[0:1, :] * rn
        var = st_ref[1:2, :] * rn - mean * mean
        inv = lax.rsqrt(var + 1e-5)
        hu = (agg_ref[...] - mean) * inv * gb_ref[0:1, :] + gb_ref[1:2, :]
        res = h_ref[:, :dres].astype(jnp.float32)
        if dres < dnew:
            res = jnp.concatenate(
                [res, jnp.zeros((bn, dnew - dres), jnp.float32)], axis=1)
        o_ref[...] = (hu[:, :dnew] + res).astype(out_dtype)

    return pl.pallas_call(
        body,
        grid=(n // bn,),
        in_specs=[
            pl.BlockSpec((bn, d), lambda i: (i, 0)),
            pl.BlockSpec((8, d), lambda i: (0, 0)),
            pl.BlockSpec((8, d), lambda i: (0, 0)),
            pl.BlockSpec((bn, dold), lambda i: (i, 0)),
        ],
        out_specs=pl.BlockSpec((bn, dnew), lambda i: (i, 0)),
        out_shape=jax.ShapeDtypeStruct((n, dnew), out_dtype),
    )(agg, stats, gb, h_old)


def _pool(h, batch, num_graphs):
    n, d = h.shape
    bn = _pick_block(n, [2000, 1000, 500, 8])
    nblk = n // bn
    b3 = batch.reshape(nblk, 1, bn)

    def body(h_ref, b_ref, o_ref):
        i = pl.program_id(0)
        b = b_ref[0]
        gi = lax.broadcasted_iota(jnp.int32, (num_graphs, 1), 0)
        oh = (b == gi).astype(jnp.float32)
        contrib = jnp.dot(oh, h_ref[...], preferred_element_type=jnp.float32)

        @pl.when(i == 0)
        def _():
            o_ref[...] = jnp.zeros_like(o_ref)

        o_ref[...] += contrib

    return pl.pallas_call(
        body,
        grid=(nblk,),
        in_specs=[
            pl.BlockSpec((bn, d), lambda i: (i, 0)),
            pl.BlockSpec((1, 1, bn), lambda i: (i, 0, 0)),
        ],
        out_specs=pl.BlockSpec((num_graphs, d), lambda i: (0, 0)),
        out_shape=jax.ShapeDtypeStruct((num_graphs, d), jnp.float32),
    )(h, b3)


def _prep_wtp(wtp_raw, dk, dm, packs):
    din, _, dout = wtp_raw.shape
    w = jnp.transpose(wtp_raw, (1, 0, 2))
    w = jnp.pad(w, ((0, 0), (0, dk - din), (0, dm - dout)))
    out = []
    s0 = 0
    for pack in packs:
        g = w[s0:s0 + pack]
        out.append(jnp.transpose(g, (1, 0, 2)).reshape(dk, pack * dm))
        s0 += pack
    assert s0 == SH
    return out


def kernel(x, pos, params, edge_index, batch):
    n_nodes = x.shape[0]
    ne = edge_index.shape[1]
    num_graphs = 8

    ei0 = edge_index[0].astype(jnp.int32)
    ei1 = edge_index[1].astype(jnp.int32)
    nch = 5
    nh = ne // nch
    chunks = []
    for c in range(nch):
        sl = slice(c * nh, (c + 1) * nh)
        chunks.append((jnp.concatenate([ei0[sl], ei1[sl]]),
                       jnp.concatenate([ei1[sl], ei0[sl]])))

    pos8 = jnp.pad(pos, ((0, 0), (0, 8 - pos.shape[1])))
    b_emb = jnp.zeros((8, EMB), jnp.float32).at[0].set(params["b_emb"])
    h = _embed(x, params["W_emb"], b_emb, pos8)

    layer_cfg = [
        (EMB, 160, 256, (1,) * SH, EMB, 256),
        (160, 160, 256, (1,) * SH, 256, 256),
        (160, 32, 128, (1,) * SH, EMB, EMB),
    ]
    cinv = None
    ea_ch = [None] * nch
    rbf_ch = [None] * nch
    for li, (lp, (dk, dm, dw, packs, dres, dnew)) in enumerate(
            zip(params["layers"], layer_cfg)):
        dout = lp["Wtp"].shape[2]
        w1 = lp["W1"]
        b1 = jnp.zeros((8, w1.shape[1]), jnp.float32).at[0].set(lp["b1"])
        w2 = jnp.pad(lp["W2"], ((0, 0), (0, dm - dout)))
        b2 = jnp.zeros((8, dm), jnp.float32).at[0].set(
            jnp.pad(lp["b2"], (0, dm - dout)))
        wtps = _prep_wtp(lp["Wtp"], dk, dm, packs)

        accs = []
        for c, (eif_c, dst_c) in enumerate(chunks):
            hs2 = _sc_gather(h, eif_c).reshape(2, nh, h.shape[1])
            if li == 0:
                msg, ea_ch[c], rbf_ch[c] = _edge_layer(
                    hs2, w1, b1, w2, b2, wtps, packs, dout, dk, dm, dw)
            else:
                msg = _edge_layer(hs2, w1, b1, w2, b2, wtps, packs,
                                  dout, dk, dm, dw, ea=ea_ch[c], rbf=rbf_ch[c])
            accs.append(_sc_scatter_add(msg.reshape(2 * nh, dw), dst_c, n_nodes))
        agg, stats, cinv = _norm_stats(accs, dout, cinv)

        gb = jnp.zeros((8, dw), jnp.float32)
        gb = gb.at[0, :dout].set(lp["gamma"])
        gb = gb.at[1, :dout].set(lp["beta"])
        h = _norm_apply(agg, stats, gb, h, dres, dnew)

    node_embedding = h
    graph_embedding = _pool(h, batch.astype(jnp.int32), num_graphs)
    return node_embedding, graph_embedding
